# baseline (speedup 1.0000x reference)
import functools
import jax
import jax.numpy as jnp
from jax.experimental import pallas as pl
from jax.experimental.pallas import tpu as pltpu

QP = 64
C = 64
NPARAM = 11


CHUNK = 256


def _body(idx_ref, p_ref, x_ref, o_ref):
    del idx_ref
    p = p_ref[0]

    def col(i):
        return p[:, i].reshape(C, 1)

    sp = [jax.nn.softplus(col(i)) for i in (0, 3, 6, 9)]
    ta = [jnp.tanh(col(i)) for i in (2, 5, 8)]
    bi = [col(i) for i in (1, 4, 7, 10)]

    L = x_ref.shape[2]

    def step(j, _):
        y = x_ref[0, :, pl.ds(j * CHUNK, CHUNK)]
        for layer in range(3):
            y = y * sp[layer] + bi[layer]
            y = y + jnp.tanh(y) * ta[layer]
        y = y * sp[3] + bi[3]
        o_ref[0, :, pl.ds(j * CHUNK, CHUNK)] = jax.nn.sigmoid(y)
        return 0

    jax.lax.fori_loop(0, L // CHUNK, step, 0)


@jax.jit
def kernel(x, index, h1, b1, a1, h2, b2, a2, h3, b3, a3, h4, b4):
    B, Cx, H, W = x.shape
    HW = H * W
    xf = x.reshape(B, Cx, HW)
    table = jnp.stack(
        [t.reshape(QP, C) for t in (h1, b1, a1, h2, b2, a2, h3, b3, a3, h4, b4)],
        axis=2,
    )

    S = 1
    L = HW // S
    grid_spec = pltpu.PrefetchScalarGridSpec(
        num_scalar_prefetch=1,
        grid=(B, S),
        in_specs=[
            pl.BlockSpec((1, C, NPARAM), lambda b, s, idx: (idx[b], 0, 0)),
            pl.BlockSpec((1, Cx, L), lambda b, s, idx: (b, 0, s)),
        ],
        out_specs=pl.BlockSpec((1, Cx, L), lambda b, s, idx: (b, 0, s)),
    )
    out = pl.pallas_call(
        _body,
        grid_spec=grid_spec,
        out_shape=jax.ShapeDtypeStruct((B, Cx, HW), x.dtype),
    )(index, table, xf)
    return out.reshape(B, Cx, H, W)


# speedup vs baseline: 1.5510x; 1.5510x over previous
import functools
import jax
import jax.numpy as jnp
from jax.experimental import pallas as pl
from jax.experimental.pallas import tpu as pltpu

QP = 64
C = 64
NPARAM = 11


CHUNK = 512


def _body(idx_ref, p_ref, x_ref, o_ref):
    del idx_ref
    p = p_ref[0]

    def col(i):
        return jnp.broadcast_to(p[:, i].reshape(C, 1), (C, CHUNK))

    sp = [jax.nn.softplus(col(i)) for i in (0, 3, 6, 9)]
    ta = [jnp.tanh(col(i)) for i in (2, 5, 8)]
    bi = [col(i) for i in (1, 4, 7, 10)]

    L = x_ref.shape[2]

    def step(j, _):
        y = x_ref[0, :, pl.ds(j * CHUNK, CHUNK)]
        for layer in range(3):
            y = y * sp[layer] + bi[layer]
            y = y + jnp.tanh(y) * ta[layer]
        y = y * sp[3] + bi[3]
        o_ref[0, :, pl.ds(j * CHUNK, CHUNK)] = 0.5 + 0.5 * jnp.tanh(0.5 * y)
        return 0

    jax.lax.fori_loop(0, L // CHUNK, step, 0, unroll=2)


@jax.jit
def kernel(x, index, h1, b1, a1, h2, b2, a2, h3, b3, a3, h4, b4):
    B, Cx, H, W = x.shape
    HW = H * W
    xf = x.reshape(B, Cx, HW)
    table = jnp.stack(
        [t.reshape(QP, C) for t in (h1, b1, a1, h2, b2, a2, h3, b3, a3, h4, b4)],
        axis=2,
    )

    S = 1
    L = HW // S
    grid_spec = pltpu.PrefetchScalarGridSpec(
        num_scalar_prefetch=1,
        grid=(B, S),
        in_specs=[
            pl.BlockSpec((1, C, NPARAM), lambda b, s, idx: (idx[b], 0, 0)),
            pl.BlockSpec((1, Cx, L), lambda b, s, idx: (b, 0, s)),
        ],
        out_specs=pl.BlockSpec((1, Cx, L), lambda b, s, idx: (b, 0, s)),
    )
    out = pl.pallas_call(
        _body,
        grid_spec=grid_spec,
        out_shape=jax.ShapeDtypeStruct((B, Cx, HW), x.dtype),
    )(index, table, xf)
    return out.reshape(B, Cx, H, W)


# speedup vs baseline: 3.6343x; 2.3433x over previous
import jax
import jax.numpy as jnp
from jax.experimental import pallas as pl
from jax.experimental.pallas import tpu as pltpu

QP = 64
C = 64
NPARAM = 11
_H_ROWS = (0, 3, 6, 9)
_A_ROWS = (2, 5, 8)


def _transform_body(t_ref, o_ref):
    for i in range(NPARAM):
        row = t_ref[:, i, :]
        if i in _H_ROWS:
            o_ref[:, i, :] = jax.nn.softplus(row)
        elif i in _A_ROWS:
            o_ref[:, i, :] = jnp.tanh(row)
        else:
            o_ref[:, i, :] = row


def _main_body(idx_ref, p_ref, x_ref, o_ref):
    del idx_ref
    p = p_ref[0]

    def row(i):
        return p[i].reshape(1, C, 1, 1)

    y = x_ref[...]
    for layer in range(3):
        y = y * row(3 * layer) + row(3 * layer + 1)
        y = y + jnp.tanh(y) * row(3 * layer + 2)
    y = y * row(9) + row(10)
    o_ref[...] = 0.5 + 0.5 * jnp.tanh(0.5 * y)


@jax.jit
def kernel(x, index, h1, b1, a1, h2, b2, a2, h3, b3, a3, h4, b4):
    B, Cx, H, W = x.shape
    table = jnp.stack(
        [t.reshape(QP, C) for t in (h1, b1, a1, h2, b2, a2, h3, b3, a3, h4, b4)],
        axis=1,
    )

    ttable = pl.pallas_call(
        _transform_body,
        out_shape=jax.ShapeDtypeStruct((QP, NPARAM, C), x.dtype),
    )(table)

    grid_spec = pltpu.PrefetchScalarGridSpec(
        num_scalar_prefetch=1,
        grid=(B,),
        in_specs=[
            pl.BlockSpec((1, NPARAM, C), lambda b, idx: (idx[b], 0, 0)),
            pl.BlockSpec((1, Cx, H, W), lambda b, idx: (b, 0, 0, 0)),
        ],
        out_specs=pl.BlockSpec((1, Cx, H, W), lambda b, idx: (b, 0, 0, 0)),
    )
    return pl.pallas_call(
        _main_body,
        grid_spec=grid_spec,
        out_shape=jax.ShapeDtypeStruct(x.shape, x.dtype),
    )(index, ttable, x)


# speedup vs baseline: 4.4408x; 1.2219x over previous
import jax
import jax.numpy as jnp
from jax.experimental import pallas as pl
from jax.experimental.pallas import tpu as pltpu

QP = 64
C = 64
NPARAM = 11
_H_ROWS = (0, 3, 6, 9)
_A_ROWS = (2, 5, 8)


def _transform_body(t_ref, o_ref):
    for i in range(NPARAM):
        row = t_ref[:, i, :]
        if i in _H_ROWS:
            o_ref[:, i, :] = jax.nn.softplus(row)
        elif i in _A_ROWS:
            o_ref[:, i, :] = jnp.tanh(row)
        else:
            o_ref[:, i, :] = row


def _main_body(idx_ref, p_ref, x_ref, o_ref):
    del idx_ref

    def chan(c, _):
        y = x_ref[0, c]

        def s(i):
            return p_ref[0, i, c]

        for layer in range(3):
            y = y * s(3 * layer) + s(3 * layer + 1)
            y = y + jnp.tanh(y) * s(3 * layer + 2)
        y = y * s(9) + s(10)
        o_ref[0, c] = 0.5 + 0.5 * jnp.tanh(0.5 * y)
        return 0

    jax.lax.fori_loop(0, C, chan, 0, unroll=2)


@jax.jit
def kernel(x, index, h1, b1, a1, h2, b2, a2, h3, b3, a3, h4, b4):
    B, Cx, H, W = x.shape
    table = jnp.stack(
        [t.reshape(QP, C) for t in (h1, b1, a1, h2, b2, a2, h3, b3, a3, h4, b4)],
        axis=1,
    )

    ttable = pl.pallas_call(
        _transform_body,
        out_shape=jax.ShapeDtypeStruct((QP, NPARAM, C), x.dtype),
    )(table)

    grid_spec = pltpu.PrefetchScalarGridSpec(
        num_scalar_prefetch=1,
        grid=(B,),
        in_specs=[
            pl.BlockSpec(
                (1, NPARAM, C),
                lambda b, idx: (idx[b], 0, 0),
                memory_space=pltpu.SMEM,
            ),
            pl.BlockSpec((1, Cx, H, W), lambda b, idx: (b, 0, 0, 0)),
        ],
        out_specs=pl.BlockSpec((1, Cx, H, W), lambda b, idx: (b, 0, 0, 0)),
    )
    return pl.pallas_call(
        _main_body,
        grid_spec=grid_spec,
        out_shape=jax.ShapeDtypeStruct(x.shape, x.dtype),
    )(index, ttable, x)


# speedup vs baseline: 4.8073x; 1.0825x over previous
import jax
import jax.numpy as jnp
from jax.experimental import pallas as pl
from jax.experimental.pallas import tpu as pltpu

QP = 64
C = 64
NPARAM = 11
_H_ROWS = (0, 3, 6, 9)
_A_ROWS = (2, 5, 8)


def _transform_body(t_ref, o_ref):
    for i in range(NPARAM):
        row = t_ref[:, i, :]
        if i in _H_ROWS:
            o_ref[:, i, :] = jax.nn.softplus(row)
        elif i in _A_ROWS:
            o_ref[:, i, :] = jnp.tanh(row)
        else:
            o_ref[:, i, :] = row


def _main_body(idx_ref, p_ref, x_ref, o_ref):
    del idx_ref

    def chan(c, _):
        y = x_ref[0, c]

        def s(i):
            return p_ref[0, i, c]

        for layer in range(3):
            y = y * s(3 * layer) + s(3 * layer + 1)
            y = y + jnp.tanh(y) * s(3 * layer + 2)
        y = y * s(9) + s(10)
        o_ref[0, c] = 0.5 + 0.5 * jnp.tanh(0.5 * y)
        return 0

    jax.lax.fori_loop(0, C, chan, 0, unroll=4)


@jax.jit
def kernel(x, index, h1, b1, a1, h2, b2, a2, h3, b3, a3, h4, b4):
    B, Cx, H, W = x.shape
    table = jnp.stack(
        [t.reshape(QP, C) for t in (h1, b1, a1, h2, b2, a2, h3, b3, a3, h4, b4)],
        axis=1,
    )

    ttable = pl.pallas_call(
        _transform_body,
        out_shape=jax.ShapeDtypeStruct((QP, NPARAM, C), x.dtype),
    )(table)

    grid_spec = pltpu.PrefetchScalarGridSpec(
        num_scalar_prefetch=1,
        grid=(B,),
        in_specs=[
            pl.BlockSpec(
                (1, NPARAM, C),
                lambda b, idx: (idx[b], 0, 0),
                memory_space=pltpu.SMEM,
            ),
            pl.BlockSpec((1, Cx, H, W), lambda b, idx: (b, 0, 0, 0)),
        ],
        out_specs=pl.BlockSpec((1, Cx, H, W), lambda b, idx: (b, 0, 0, 0)),
    )
    return pl.pallas_call(
        _main_body,
        grid_spec=grid_spec,
        out_shape=jax.ShapeDtypeStruct(x.shape, x.dtype),
    )(index, ttable, x)


# speedup vs baseline: 4.9145x; 1.0223x over previous
import jax
import jax.numpy as jnp
from jax.experimental import pallas as pl
from jax.experimental.pallas import tpu as pltpu

QP = 64
C = 64
NPARAM = 11
_H_ROWS = (0, 3, 6, 9)
_A_ROWS = (2, 5, 8)


def _transform_body(t_ref, o_ref):
    for i in range(NPARAM):
        row = t_ref[:, i, :]
        if i in _H_ROWS:
            o_ref[:, i, :] = jax.nn.softplus(row)
        elif i in _A_ROWS:
            o_ref[:, i, :] = jnp.tanh(row)
        else:
            o_ref[:, i, :] = row


def _main_body(idx_ref, p_ref, x_ref, o_ref):
    del idx_ref

    def chan(c, _):
        y = x_ref[0, c]

        def s(i):
            return p_ref[0, i, c]

        for layer in range(3):
            y = y * s(3 * layer) + s(3 * layer + 1)
            y = y + jnp.tanh(y) * s(3 * layer + 2)
        y = y * s(9) + s(10)
        o_ref[0, c] = 0.5 + 0.5 * jnp.tanh(0.5 * y)
        return 0

    jax.lax.fori_loop(0, C, chan, 0, unroll=8)


@jax.jit
def kernel(x, index, h1, b1, a1, h2, b2, a2, h3, b3, a3, h4, b4):
    B, Cx, H, W = x.shape
    table = jnp.stack(
        [t.reshape(QP, C) for t in (h1, b1, a1, h2, b2, a2, h3, b3, a3, h4, b4)],
        axis=1,
    )

    ttable = pl.pallas_call(
        _transform_body,
        out_shape=jax.ShapeDtypeStruct((QP, NPARAM, C), x.dtype),
    )(table)

    grid_spec = pltpu.PrefetchScalarGridSpec(
        num_scalar_prefetch=1,
        grid=(B,),
        in_specs=[
            pl.BlockSpec(
                (1, NPARAM, C),
                lambda b, idx: (idx[b], 0, 0),
                memory_space=pltpu.SMEM,
            ),
            pl.BlockSpec((1, Cx, H, W), lambda b, idx: (b, 0, 0, 0)),
        ],
        out_specs=pl.BlockSpec((1, Cx, H, W), lambda b, idx: (b, 0, 0, 0)),
    )
    return pl.pallas_call(
        _main_body,
        grid_spec=grid_spec,
        out_shape=jax.ShapeDtypeStruct(x.shape, x.dtype),
    )(index, ttable, x)


# speedup vs baseline: 5.0372x; 1.0250x over previous
import jax
import jax.numpy as jnp
from jax.experimental import pallas as pl
from jax.experimental.pallas import tpu as pltpu

QP = 64
C = 64
NPARAM = 11
_H_ROWS = (0, 3, 6, 9)
_A_ROWS = (2, 5, 8)


def _transform_body(t_ref, o_ref):
    sp = [jax.nn.softplus(t_ref[:, i, :]) for i in (0, 3, 6, 9)]
    ta = [jnp.tanh(t_ref[:, i, :]) for i in (2, 5, 8)]
    b = [t_ref[:, i, :] for i in (1, 4, 7, 10)]
    o_ref[:, 0, :] = sp[0]
    o_ref[:, 1, :] = b[0]
    for layer in range(3):
        scale = sp[layer + 1] if layer < 2 else 0.5 * sp[3]
        o_ref[:, 3 * layer + 2, :] = scale
        o_ref[:, 3 * layer + 3, :] = ta[layer] * scale
        o_ref[:, 3 * layer + 4, :] = b[layer + 1] * (1.0 if layer < 2 else 0.5)


def _main_body(idx_ref, p_ref, x_ref, o_ref):
    del idx_ref

    def chan(c, _):
        y = x_ref[0, c]

        def s(i):
            return p_ref[0, i, c]

        y = y * s(0) + s(1)
        for layer in range(3):
            y = y * s(3 * layer + 2) + jnp.tanh(y) * s(3 * layer + 3) + s(3 * layer + 4)
        o_ref[0, c] = 0.5 * jnp.tanh(y) + 0.5
        return 0

    jax.lax.fori_loop(0, C, chan, 0, unroll=8)


@jax.jit
def kernel(x, index, h1, b1, a1, h2, b2, a2, h3, b3, a3, h4, b4):
    B, Cx, H, W = x.shape
    table = jnp.stack(
        [t.reshape(QP, C) for t in (h1, b1, a1, h2, b2, a2, h3, b3, a3, h4, b4)],
        axis=1,
    )

    ttable = pl.pallas_call(
        _transform_body,
        out_shape=jax.ShapeDtypeStruct((QP, NPARAM, C), x.dtype),
    )(table)

    grid_spec = pltpu.PrefetchScalarGridSpec(
        num_scalar_prefetch=1,
        grid=(B,),
        in_specs=[
            pl.BlockSpec(
                (1, NPARAM, C),
                lambda b, idx: (idx[b], 0, 0),
                memory_space=pltpu.SMEM,
            ),
            pl.BlockSpec((1, Cx, H, W), lambda b, idx: (b, 0, 0, 0)),
        ],
        out_specs=pl.BlockSpec((1, Cx, H, W), lambda b, idx: (b, 0, 0, 0)),
    )
    return pl.pallas_call(
        _main_body,
        grid_spec=grid_spec,
        out_shape=jax.ShapeDtypeStruct(x.shape, x.dtype),
    )(index, ttable, x)
